# baseline (speedup 1.0000x reference)
import functools

import jax
import jax.numpy as jnp
from jax import lax
from jax.experimental import pallas as pl
from jax.experimental.pallas import tpu as pltpu
from jax.experimental.pallas import tpu_sc as plsc

B = 1024
N_MF = 4096
N_GC = 8192
NSYN = 4
THETA = 0.75

L = 16
ROWS_PER_BLK = 16
GC_CHUNK = 2048


def _sc_body(x_hbm, idxt_hbm, wt_hbm, out_hbm, xbuf, ibuf, wbuf, obuf):
    nc = 2
    wid = lax.axis_index("s") * nc + lax.axis_index("c")
    rows_per_worker = B // 32

    n_groups = GC_CHUNK // L

    for rb in range(rows_per_worker // ROWS_PER_BLK):
        row0 = wid * rows_per_worker + rb * ROWS_PER_BLK
        pltpu.sync_copy(x_hbm.at[pl.ds(row0, ROWS_PER_BLK)], xbuf)
        for c in range(N_GC // GC_CHUNK):
            pltpu.sync_copy(idxt_hbm.at[:, pl.ds(c * GC_CHUNK, GC_CHUNK)], ibuf)
            pltpu.sync_copy(wt_hbm.at[:, pl.ds(c * GC_CHUNK, GC_CHUNK)], wbuf)

            def group_body(g, _):
                g16 = pl.multiple_of(g * L, L)
                iv = [ibuf[s, pl.ds(g16, L)] for s in range(NSYN)]
                wv = [wbuf[s, pl.ds(g16, L)] for s in range(NSYN)]

                for r in range(ROWS_PER_BLK):
                    rv = jnp.full((L,), r, dtype=jnp.int32)
                    acc = plsc.load_gather(xbuf, [rv, iv[0]]) * wv[0]
                    for s in range(1, NSYN):
                        acc = acc + plsc.load_gather(xbuf, [rv, iv[s]]) * wv[s]
                    res = jnp.maximum(acc - THETA, 0.0)
                    obuf[r, pl.ds(g16, L)] = res
                return 0

            lax.fori_loop(0, n_groups, group_body, 0)
            pltpu.sync_copy(
                obuf,
                out_hbm.at[pl.ds(row0, ROWS_PER_BLK), pl.ds(c * GC_CHUNK, GC_CHUNK)],
            )


@jax.jit
def _mossy_granule_sc(x, idx_t, w_t):
    mesh = plsc.VectorSubcoreMesh(core_axis_name="c", subcore_axis_name="s")
    kern = pl.kernel(
        _sc_body,
        out_type=jax.ShapeDtypeStruct((B, N_GC), jnp.float32),
        mesh=mesh,
        compiler_params=pltpu.CompilerParams(
            use_tc_tiling_on_sc=False, needs_layout_passes=False
        ),
        scratch_types=[
            pltpu.VMEM((ROWS_PER_BLK, N_MF), jnp.float32),
            pltpu.VMEM((NSYN, GC_CHUNK), jnp.int32),
            pltpu.VMEM((NSYN, GC_CHUNK), jnp.float32),
            pltpu.VMEM((ROWS_PER_BLK, GC_CHUNK), jnp.float32),
        ],
    )
    return kern(x, idx_t, w_t)


def kernel(x, idx, W_conn):
    idx_t = idx.T.astype(jnp.int32)
    w_t = W_conn.T.astype(jnp.float32)
    return _mossy_granule_sc(x, idx_t, w_t)


# speedup vs baseline: 1.5789x; 1.5789x over previous
import functools

import jax
import jax.numpy as jnp
from jax import lax
from jax.experimental import pallas as pl
from jax.experimental.pallas import tpu as pltpu
from jax.experimental.pallas import tpu_sc as plsc

B = 1024
N_MF = 4096
N_GC = 8192
NSYN = 4
THETA = 0.75

L = 16
ROWS_PER_BLK = 16
GC_CHUNK = 2048


def _sc_body(x_hbm, idxt_hbm, wt_hbm, out_hbm, xbuf, ibuf, wbuf, obuf):
    nc = 2
    wid = lax.axis_index("s") * nc + lax.axis_index("c")
    rows_per_worker = B // 32

    n_groups = GC_CHUNK // L

    for rb in range(rows_per_worker // ROWS_PER_BLK):
        row0 = wid * rows_per_worker + rb * ROWS_PER_BLK
        pltpu.sync_copy(x_hbm.at[pl.ds(row0, ROWS_PER_BLK)], xbuf)
        for c in range(N_GC // GC_CHUNK):
            pltpu.sync_copy(idxt_hbm.at[:, pl.ds(c * GC_CHUNK, GC_CHUNK)], ibuf)
            pltpu.sync_copy(wt_hbm.at[:, pl.ds(c * GC_CHUNK, GC_CHUNK)], wbuf)

            def group_body(g, _):
                g16 = pl.multiple_of(g * L, L)
                iv = [ibuf[s, pl.ds(g16, L)] for s in range(NSYN)]
                wv = [wbuf[s, pl.ds(g16, L)] for s in range(NSYN)]

                RGRP = 4
                for r0 in range(0, ROWS_PER_BLK, RGRP):
                    gath = []
                    for r in range(r0, r0 + RGRP):
                        rv = jnp.full((L,), r, dtype=jnp.int32)
                        gath.append(
                            [plsc.load_gather(xbuf, [rv, iv[s]]) for s in range(NSYN)]
                        )
                    for k, r in enumerate(range(r0, r0 + RGRP)):
                        ga = gath[k]
                        acc = (ga[0] * wv[0] + ga[1] * wv[1]) + (
                            ga[2] * wv[2] + ga[3] * wv[3]
                        )
                        obuf[r, pl.ds(g16, L)] = jnp.maximum(acc - THETA, 0.0)
                return 0

            lax.fori_loop(0, n_groups, group_body, 0)
            pltpu.sync_copy(
                obuf,
                out_hbm.at[pl.ds(row0, ROWS_PER_BLK), pl.ds(c * GC_CHUNK, GC_CHUNK)],
            )


@jax.jit
def _mossy_granule_sc(x, idx_t, w_t):
    mesh = plsc.VectorSubcoreMesh(core_axis_name="c", subcore_axis_name="s")
    kern = pl.kernel(
        _sc_body,
        out_type=jax.ShapeDtypeStruct((B, N_GC), jnp.float32),
        mesh=mesh,
        compiler_params=pltpu.CompilerParams(
            use_tc_tiling_on_sc=False, needs_layout_passes=False
        ),
        scratch_types=[
            pltpu.VMEM((ROWS_PER_BLK, N_MF), jnp.float32),
            pltpu.VMEM((NSYN, GC_CHUNK), jnp.int32),
            pltpu.VMEM((NSYN, GC_CHUNK), jnp.float32),
            pltpu.VMEM((ROWS_PER_BLK, GC_CHUNK), jnp.float32),
        ],
    )
    return kern(x, idx_t, w_t)


def kernel(x, idx, W_conn):
    idx_t = idx.T.astype(jnp.int32)
    w_t = W_conn.T.astype(jnp.float32)
    return _mossy_granule_sc(x, idx_t, w_t)


# speedup vs baseline: 1.8166x; 1.1505x over previous
import functools

import jax
import jax.numpy as jnp
from jax import lax
from jax.experimental import pallas as pl
from jax.experimental.pallas import tpu as pltpu
from jax.experimental.pallas import tpu_sc as plsc

B = 1024
N_MF = 4096
N_GC = 8192
NSYN = 4
THETA = 0.75

L = 16
ROWS_PER_BLK = 16
GC_CHUNK = 2048


def _sc_body(x_hbm, idxt_hbm, wt_hbm, out_hbm, xbuf, ibuf, wbuf, obuf):
    nc = 2
    wid = lax.axis_index("s") * nc + lax.axis_index("c")
    rows_per_worker = B // 32

    n_groups = GC_CHUNK // L

    for rb in range(rows_per_worker // ROWS_PER_BLK):
        row0 = wid * rows_per_worker + rb * ROWS_PER_BLK
        pltpu.sync_copy(x_hbm.at[pl.ds(row0, ROWS_PER_BLK)], xbuf)
        for c in range(N_GC // GC_CHUNK):
            pltpu.sync_copy(idxt_hbm.at[:, pl.ds(c * GC_CHUNK, GC_CHUNK)], ibuf)
            pltpu.sync_copy(wt_hbm.at[:, pl.ds(c * GC_CHUNK, GC_CHUNK)], wbuf)

            @plsc.parallel_loop(0, n_groups, 1)
            def group_body(g):
                g16 = pl.multiple_of(g * L, L)
                iv = [ibuf[s, pl.ds(g16, L)] for s in range(NSYN)]
                wv = [wbuf[s, pl.ds(g16, L)] for s in range(NSYN)]

                RGRP = 4
                for r0 in range(0, ROWS_PER_BLK, RGRP):
                    gath = []
                    for r in range(r0, r0 + RGRP):
                        rv = jnp.full((L,), r, dtype=jnp.int32)
                        gath.append(
                            [plsc.load_gather(xbuf, [rv, iv[s]]) for s in range(NSYN)]
                        )
                    for k, r in enumerate(range(r0, r0 + RGRP)):
                        ga = gath[k]
                        acc = (ga[0] * wv[0] + ga[1] * wv[1]) + (
                            ga[2] * wv[2] + ga[3] * wv[3]
                        )
                        obuf[r, pl.ds(g16, L)] = jnp.maximum(acc - THETA, 0.0)

            pltpu.sync_copy(
                obuf,
                out_hbm.at[pl.ds(row0, ROWS_PER_BLK), pl.ds(c * GC_CHUNK, GC_CHUNK)],
            )


@jax.jit
def _mossy_granule_sc(x, idx_t, w_t):
    mesh = plsc.VectorSubcoreMesh(core_axis_name="c", subcore_axis_name="s")
    kern = pl.kernel(
        _sc_body,
        out_type=jax.ShapeDtypeStruct((B, N_GC), jnp.float32),
        mesh=mesh,
        compiler_params=pltpu.CompilerParams(
            use_tc_tiling_on_sc=False, needs_layout_passes=False
        ),
        scratch_types=[
            pltpu.VMEM((ROWS_PER_BLK, N_MF), jnp.float32),
            pltpu.VMEM((NSYN, GC_CHUNK), jnp.int32),
            pltpu.VMEM((NSYN, GC_CHUNK), jnp.float32),
            pltpu.VMEM((ROWS_PER_BLK, GC_CHUNK), jnp.float32),
        ],
    )
    return kern(x, idx_t, w_t)


def kernel(x, idx, W_conn):
    idx_t = idx.T.astype(jnp.int32)
    w_t = W_conn.T.astype(jnp.float32)
    return _mossy_granule_sc(x, idx_t, w_t)
